# baseline (speedup 1.0000x reference)
import jax
import jax.numpy as jnp
from jax import lax
from jax.experimental import pallas as pl
from jax.experimental.pallas import tpu as pltpu

N = 1024
C = 100000
W = 128


NSEM = 8
B = N // NSEM


def _tc_kernel(pt_ref, tgt_ref, rwd_ref, o_ref, v_ref, *sems):
    copies = []
    for i in range(N):
        t = tgt_ref[i]
        cb = (i // W) * W
        copies.append(pltpu.make_async_copy(
            pt_ref.at[pl.ds(t, 1), pl.ds(cb, W)],
            v_ref.at[pl.ds(i, 1), :], sems[i // B]))
    for c in copies:
        c.start()
    for g in range(NSEM):
        pltpu.make_async_copy(
            pt_ref.at[pl.ds(0, B), pl.ds(0, W)],
            v_ref.at[pl.ds(g * B, B), :], sems[g]).wait()

    lane_want = lax.broadcasted_iota(jnp.int32, (N, W), 0) & (W - 1)
    lane = lax.broadcasted_iota(jnp.int32, (N, W), 1)
    sel = jnp.where(lane == lane_want, v_ref[...], 0.0)
    o_ref[0, 0] = -jnp.sum(sel * rwd_ref[...])


@jax.jit
def kernel(prob, target, reward):
    pt = prob.T
    loss = pl.pallas_call(
        _tc_kernel,
        out_shape=jax.ShapeDtypeStruct((1, 1), jnp.float32),
        in_specs=[pl.BlockSpec(memory_space=pl.ANY),
                  pl.BlockSpec(memory_space=pltpu.SMEM),
                  pl.BlockSpec(memory_space=pltpu.VMEM)],
        out_specs=pl.BlockSpec(memory_space=pltpu.SMEM),
        scratch_shapes=[pltpu.VMEM((N, W), jnp.float32)]
        + [pltpu.SemaphoreType.DMA] * NSEM,
    )(pt, target, reward.reshape(N, 1))
    return loss[0, 0]


# speedup vs baseline: 1.3433x; 1.3433x over previous
import jax
import jax.numpy as jnp
from jax import lax
from jax.experimental import pallas as pl
from jax.experimental.pallas import tpu as pltpu

N = 1024
C = 100000
W = 128


def _tc_kernel(pt_ref, tgt_ref, rwd_ref, o_ref, v_ref, sem):
    copies = []
    for i in range(N):
        t = tgt_ref[i]
        cb = (i // W) * W
        copies.append(pltpu.make_async_copy(
            pt_ref.at[pl.ds(t, 1), pl.ds(cb, W)],
            v_ref.at[pl.ds(i, 1), :], sem))
    for c in copies:
        c.start()
    pltpu.make_async_copy(
        pt_ref.at[pl.ds(0, N), pl.ds(0, W)], v_ref, sem).wait()

    v3 = v_ref[...].reshape(N // W, W, W)
    j = lax.broadcasted_iota(jnp.int32, (N // W, W, W), 1)
    lane = lax.broadcasted_iota(jnp.int32, (N // W, W, W), 2)
    m = jnp.sum(jnp.where(lane == j, v3, 0.0), axis=1)
    o_ref[0, 0] = -jnp.sum(m * rwd_ref[...])


@jax.jit
def kernel(prob, target, reward):
    pt = prob.T
    loss = pl.pallas_call(
        _tc_kernel,
        out_shape=jax.ShapeDtypeStruct((1, 1), jnp.float32),
        in_specs=[pl.BlockSpec(memory_space=pl.ANY),
                  pl.BlockSpec(memory_space=pltpu.SMEM),
                  pl.BlockSpec(memory_space=pltpu.VMEM)],
        out_specs=pl.BlockSpec(memory_space=pltpu.SMEM),
        scratch_shapes=[pltpu.VMEM((N, W), jnp.float32),
                        pltpu.SemaphoreType.DMA],
    )(pt, target, reward.reshape(N // W, W))
    return loss[0, 0]
